# baseline (speedup 1.0000x reference)
import functools

import jax
import jax.numpy as jnp
from jax import lax
from jax.experimental import pallas as pl
from jax.experimental.pallas import tpu as pltpu
from jax.experimental.pallas import tpu_sc as plsc

VOCAB = 1000
B, T = 1024, 50
N = B * T
NC, NS, L = 2, 16, 16
NW = NC * NS
PER_W = N // NW
K = 16
NB = 3
LAG = 1
NCHUNK = PER_W // K
NITER = NCHUNK // NB
REM = NCHUNK - NITER * NB


def _lse_body(tab_ref, out_ref):
    x = tab_ref[...]
    m = jnp.max(x, axis=1)
    s = jnp.sum(jnp.exp(x - m[:, None]), axis=1)
    out_ref[...] = m + jnp.log(s)


def _row_lse(table):
    return pl.pallas_call(
        _lse_body,
        out_shape=jax.ShapeDtypeStruct((VOCAB,), jnp.float32),
    )(table)


_sc_mesh = plsc.VectorSubcoreMesh(core_axis_name="c", subcore_axis_name="s")


@functools.partial(
    pl.kernel,
    mesh=_sc_mesh,
    compiler_params=pltpu.CompilerParams(use_tc_tiling_on_sc=False),
    out_type=[
        jax.ShapeDtypeStruct((N, VOCAB), jnp.float32),
        jax.ShapeDtypeStruct((NW, L), jnp.float32),
    ],
    scratch_types=[
        pltpu.VMEM((PER_W,), jnp.int32),
        pltpu.VMEM((PER_W,), jnp.int32),
        [pltpu.VMEM((K, VOCAB), jnp.float32)] * NB,
        pltpu.VMEM((PER_W,), jnp.float32),
        pltpu.VMEM((PER_W,), jnp.float32),
        pltpu.VMEM((L,), jnp.float32),
        pltpu.VMEM_SHARED((VOCAB, VOCAB), jnp.float32),
        [pltpu.SemaphoreType.DMA] * NB,
        [pltpu.SemaphoreType.DMA] * NB,
        pltpu.SemaphoreType.DMA,
        pltpu.SemaphoreType.DMA,
    ],
)
def _sc_gather_loss(table_hbm, tabflat_hbm, idx_hbm, pick_hbm, lse_hbm,
                    out_hbm, part_hbm,
                    idx_a, pick_a, rows, lsev_v, picked_v, part_v, tab_sp,
                    gsem, wsem, psem, lsem):
    wid = lax.axis_index("s") * NC + lax.axis_index("c")
    sid = lax.axis_index("s")
    base = wid * PER_W
    @pl.when(sid < 8)
    def _():
        r0 = sid * (VOCAB // 8)
        pltpu.sync_copy(table_hbm.at[pl.ds(r0, VOCAB // 8)],
                        tab_sp.at[pl.ds(r0, VOCAB // 8)])
    pltpu.sync_copy(idx_hbm.at[pl.ds(base, PER_W)], idx_a)
    pltpu.sync_copy(pick_hbm.at[pl.ds(base, PER_W)], pick_a)
    pltpu.async_copy(lse_hbm.at[idx_a], lsev_v, lsem)
    pltpu.async_copy(tabflat_hbm.at[pick_a], picked_v, psem)
    plsc.subcore_barrier()

    def gather(c, b):
        pltpu.async_copy(tab_sp.at[idx_a.at[pl.ds(c * K, K)]],
                         rows[b], gsem[b])

    def wait_gather(c, b):
        pltpu.make_async_copy(tab_sp.at[idx_a.at[pl.ds(c * K, K)]],
                              rows[b], gsem[b]).wait()

    def write(c, b):
        pltpu.async_copy(rows[b], out_hbm.at[pl.ds(base + c * K, K)],
                         wsem[b])

    def wait_write(c, b):
        pltpu.make_async_copy(rows[b], out_hbm.at[pl.ds(base + c * K, K)],
                              wsem[b]).wait()

    for g in range(LAG):
        gather(g, g)
    for c in range(NB):
        wait_gather(c, c)
        write(c, c)
        if c + LAG >= NB:
            wait_write(c + LAG - NB, (c + LAG) % NB)
        gather(c + LAG, (c + LAG) % NB)

    def step(i, carry):
        for b in range(NB):
            c = i * NB + b
            wait_gather(c, b)
            write(c, b)
            pb = (b + LAG) % NB
            wait_write(c + LAG - NB, pb)
            @pl.when(c + LAG < NCHUNK)
            def _():
                gather(c + LAG, pb)
        return carry

    lax.fori_loop(1, NITER, step, jnp.int32(0))
    for c in range(NITER * NB, NCHUNK):
        wait_gather(c, c % NB)
        write(c, c % NB)
    for c in range(NITER * NB + LAG - NB, NCHUNK):
        wait_write(c, c % NB)

    pltpu.make_async_copy(lse_hbm.at[idx_a], lsev_v, lsem).wait()
    pltpu.make_async_copy(tabflat_hbm.at[pick_a], picked_v, psem).wait()

    def red(j, acc):
        return acc + (lsev_v[pl.ds(j * L, L)] - picked_v[pl.ds(j * L, L)])

    acc = lax.fori_loop(0, PER_W // L, red, jnp.zeros((L,), jnp.float32))
    part_v[...] = acc
    pltpu.sync_copy(part_v, part_hbm.at[wid])


def kernel(idx, targets, token_embedding_table):
    table = token_embedding_table.astype(jnp.float32)
    idx32 = idx.reshape(N).astype(jnp.int32)
    tgt32 = targets.reshape(N).astype(jnp.int32)
    pick32 = idx32 * VOCAB + tgt32
    lse = _row_lse(table)
    tabflat = jnp.pad(table.reshape(VOCAB * VOCAB), (0, 8))
    logits_flat, partials = _sc_gather_loss(
        table, tabflat, idx32, pick32, lse)
    loss = jnp.sum(partials) / jnp.float32(N)
    return logits_flat.reshape(B, T, VOCAB), loss


# speedup vs baseline: 1.0060x; 1.0060x over previous
import functools

import jax
import jax.numpy as jnp
from jax import lax
from jax.experimental import pallas as pl
from jax.experimental.pallas import tpu as pltpu
from jax.experimental.pallas import tpu_sc as plsc

VOCAB = 1000
B, T = 1024, 50
N = B * T
NC, NS, L = 2, 16, 16
NW = NC * NS
PER_W = N // NW
K = 8
NB = 5
LAG = 2
NCHUNK = PER_W // K
NITER = NCHUNK // NB


def _lse_body(tab_ref, out_ref):
    x = tab_ref[...]
    m = jnp.max(x, axis=1)
    s = jnp.sum(jnp.exp(x - m[:, None]), axis=1)
    out_ref[...] = m + jnp.log(s)


def _row_lse(table):
    return pl.pallas_call(
        _lse_body,
        out_shape=jax.ShapeDtypeStruct((VOCAB,), jnp.float32),
    )(table)


def _flatten(table):
    return jnp.pad(table.reshape(VOCAB * VOCAB), (0, 8))


_sc_mesh = plsc.VectorSubcoreMesh(core_axis_name="c", subcore_axis_name="s")


@functools.partial(
    pl.kernel,
    mesh=_sc_mesh,
    compiler_params=pltpu.CompilerParams(use_tc_tiling_on_sc=False),
    out_type=[
        jax.ShapeDtypeStruct((N, VOCAB), jnp.float32),
        jax.ShapeDtypeStruct((NW, L), jnp.float32),
    ],
    scratch_types=[
        pltpu.VMEM((PER_W,), jnp.int32),
        pltpu.VMEM((PER_W,), jnp.int32),
        [pltpu.VMEM((K, VOCAB), jnp.float32)] * NB,
        pltpu.VMEM((PER_W,), jnp.float32),
        pltpu.VMEM((PER_W,), jnp.float32),
        pltpu.VMEM((L,), jnp.float32),
        pltpu.VMEM_SHARED((VOCAB, VOCAB), jnp.float32),
        [pltpu.SemaphoreType.DMA] * NB,
        [pltpu.SemaphoreType.DMA] * NB,
        pltpu.SemaphoreType.DMA,
        pltpu.SemaphoreType.DMA,
    ],
)
def _sc_gather_loss(table_hbm, tabflat_hbm, idx_hbm, pick_hbm, lse_hbm,
                    out_hbm, part_hbm,
                    idx_a, pick_a, rows, lsev_v, picked_v, part_v, tab_sp,
                    gsem, wsem, psem, lsem):
    wid = lax.axis_index("s") * NC + lax.axis_index("c")
    sid = lax.axis_index("s")
    base = wid * PER_W
    @pl.when(sid < 8)
    def _():
        r0 = sid * (VOCAB // 8)
        pltpu.sync_copy(table_hbm.at[pl.ds(r0, VOCAB // 8)],
                        tab_sp.at[pl.ds(r0, VOCAB // 8)])
    pltpu.sync_copy(idx_hbm.at[pl.ds(base, PER_W)], idx_a)
    pltpu.sync_copy(pick_hbm.at[pl.ds(base, PER_W)], pick_a)
    pltpu.async_copy(lse_hbm.at[idx_a], lsev_v, lsem)
    pltpu.async_copy(tabflat_hbm.at[pick_a], picked_v, psem)
    plsc.subcore_barrier()

    def gather(c, b):
        pltpu.async_copy(tab_sp.at[idx_a.at[pl.ds(c * K, K)]],
                         rows[b], gsem[b])

    def wait_gather(c, b):
        pltpu.make_async_copy(tab_sp.at[idx_a.at[pl.ds(c * K, K)]],
                              rows[b], gsem[b]).wait()

    def write(c, b):
        pltpu.async_copy(rows[b], out_hbm.at[pl.ds(base + c * K, K)],
                         wsem[b])

    def wait_write(c, b):
        pltpu.make_async_copy(rows[b], out_hbm.at[pl.ds(base + c * K, K)],
                              wsem[b]).wait()

    for g in range(LAG):
        gather(g, g)
    for c in range(NB):
        wait_gather(c, c)
        write(c, c)
        if c + LAG >= NB:
            wait_write(c + LAG - NB, (c + LAG) % NB)
        gather(c + LAG, (c + LAG) % NB)

    def step(i, carry):
        for b in range(NB):
            c = i * NB + b
            wait_gather(c, b)
            write(c, b)
            pb = (b + LAG) % NB
            wait_write(c + LAG - NB, pb)
            @pl.when(c + LAG < NCHUNK)
            def _():
                gather(c + LAG, pb)
        return carry

    lax.fori_loop(1, NITER, step, jnp.int32(0))
    for c in range(NCHUNK - (NB - LAG), NCHUNK):
        wait_write(c, c % NB)

    pltpu.make_async_copy(lse_hbm.at[idx_a], lsev_v, lsem).wait()
    pltpu.make_async_copy(tabflat_hbm.at[pick_a], picked_v, psem).wait()

    def red(j, acc):
        return acc + (lsev_v[pl.ds(j * L, L)] - picked_v[pl.ds(j * L, L)])

    acc = lax.fori_loop(0, PER_W // L, red, jnp.zeros((L,), jnp.float32))
    part_v[...] = acc
    pltpu.sync_copy(part_v, part_hbm.at[wid])


def kernel(idx, targets, token_embedding_table):
    table = token_embedding_table.astype(jnp.float32)
    idx32 = idx.reshape(N).astype(jnp.int32)
    tgt32 = targets.reshape(N).astype(jnp.int32)
    pick32 = idx32 * VOCAB + tgt32
    lse = _row_lse(table)
    tabflat = _flatten(table)
    logits_flat, partials = _sc_gather_loss(
        table, tabflat, idx32, pick32, lse)
    loss = jnp.sum(partials) / jnp.float32(N)
    return logits_flat.reshape(B, T, VOCAB), loss
